# baseline (speedup 1.0000x reference)
import functools
import jax
import jax.numpy as jnp
from jax import lax
from jax.experimental import pallas as pl
from jax.experimental.pallas import tpu as pltpu
from jax.experimental.pallas import tpu_sc as plsc

N_NODES = 10000
N_EDGES = 320000
D = 128

NB = 2000


def _prep_body(mem_ref, qs_ref, qr_ref, wq_ref, wk_ref,
               tvi_ref, tvj_ref, tq_ref, mrr_ref):
    f32 = jnp.float32
    mem = mem_ref[...]
    qs = qs_ref[...]
    qr = qr_ref[...]

    def blk(w, i):
        return w[:, i * D:(i + 1) * D]

    Wq0, Wq1, Wq2, Wq3 = (blk(wq_ref, i) for i in range(4))
    Wk0, Wk1, Wk2, Wk3 = (blk(wk_ref, i) for i in range(4))

    def dg(a, b):
        return lax.dot_general(a, b, (((1,), (0,)), ((), ())),
                               preferred_element_type=f32)

    def dgT(a, b):
        return lax.dot_general(a, b, (((1,), (1,)), ((), ())),
                               preferred_element_type=f32)

    def ctr(a, b):
        return lax.dot_general(a, b, (((0,), (0,)), ((), ())),
                               preferred_element_type=f32)

    C = dgT(qs, Wq2) + dgT(qr, Wq3)
    Dm = dgT(qs, Wk2) + dgT(qr, Wk3)

    u = dg(mem, ctr(Wq0, Wk0))
    g1 = dg(mem, ctr(Wq0, Wk1))
    g2 = dg(mem, ctr(Wk0, Wq1))
    d2 = dg(Dm, Wq0)
    c2 = dg(C, Wk0)
    hh = dg(Dm, Wq1) + dg(C, Wk1)
    s = jnp.sum(C * Dm, axis=-1)

    col0 = (lax.broadcasted_iota(jnp.int32, (NB, D), 1) == 0).astype(f32)
    srow = s[:, None] * col0

    tvi_ref[...] = jnp.concatenate([u, g1, mem], axis=1)
    tvj_ref[...] = jnp.concatenate([mem, g2], axis=1)
    tq_ref[...] = jnp.concatenate([d2, c2, hh, srow], axis=1)
    mrr_ref[...] = ctr(Wq1, Wk1)


def _prep(mem, qs, qr, Wq, Wk):
    grid = N_NODES // NB
    return pl.pallas_call(
        _prep_body,
        grid=(grid,),
        in_specs=[
            pl.BlockSpec((NB, D), lambda i: (i, 0)),
            pl.BlockSpec((NB, D), lambda i: (i, 0)),
            pl.BlockSpec((NB, D), lambda i: (i, 0)),
            pl.BlockSpec((4 * D, 4 * D), lambda i: (0, 0)),
            pl.BlockSpec((4 * D, 4 * D), lambda i: (0, 0)),
        ],
        out_specs=[
            pl.BlockSpec((NB, 3 * D), lambda i: (i, 0)),
            pl.BlockSpec((NB, 2 * D), lambda i: (i, 0)),
            pl.BlockSpec((NB, 4 * D), lambda i: (i, 0)),
            pl.BlockSpec((D, D), lambda i: (0, 0)),
        ],
        out_shape=[
            jax.ShapeDtypeStruct((N_NODES, 3 * D), jnp.float32),
            jax.ShapeDtypeStruct((N_NODES, 2 * D), jnp.float32),
            jax.ShapeDtypeStruct((N_NODES, 4 * D), jnp.float32),
            jax.ShapeDtypeStruct((D, D), jnp.float32),
        ],
    )(mem, qs, qr, Wq, Wk)


EB = 3200


def _trel_body(rel_ref, mrr_ref, out_ref):
    rel = rel_ref[...]
    z = lax.dot_general(rel, mrr_ref[...], (((1,), (0,)), ((), ())),
                        preferred_element_type=jnp.float32)
    out_ref[...] = jnp.sum(z * rel, axis=-1)[None, None, :]


def _trel(rel, mrr):
    grid = N_EDGES // EB
    out = pl.pallas_call(
        _trel_body,
        grid=(grid,),
        in_specs=[
            pl.BlockSpec((EB, D), lambda i: (i, 0)),
            pl.BlockSpec((D, D), lambda i: (0, 0)),
        ],
        out_specs=pl.BlockSpec((1, 1, EB), lambda i: (i, 0, 0)),
        out_shape=jax.ShapeDtypeStruct((grid, 1, EB), jnp.float32),
    )(rel, mrr)
    return out.reshape(-1)


_NC, _NS = 2, 16
_NW = _NC * _NS
_EPT = N_EDGES // _NW
_CB = 16
_NCHUNK = _EPT // _CB
_DW = D // 2


def _shuf(v, idx):
    return lax.gather(
        v, idx[:, None],
        lax.GatherDimensionNumbers(offset_dims=(), collapsed_slice_dims=(0,),
                                   start_index_map=(0,)),
        slice_sizes=(1,),
        mode=lax.GatherScatterMode.PROMISE_IN_BOUNDS)


def _sc_edges(tvi, tvj, tq, rel, trel, vi, vj, qx):
    mesh = plsc.VectorSubcoreMesh(core_axis_name="c", subcore_axis_name="s")

    @functools.partial(
        pl.kernel,
        out_type=jax.ShapeDtypeStruct((N_EDGES,), jnp.float32),
        mesh=mesh,
        compiler_params=pltpu.CompilerParams(use_tc_tiling_on_sc=False),
        scratch_types=[
            pltpu.VMEM((2, _CB), jnp.int32),
            pltpu.VMEM((2, _CB), jnp.int32),
            pltpu.VMEM((2, _CB), jnp.int32),
            pltpu.VMEM((2, _CB, 3 * D), jnp.float32),
            pltpu.VMEM((2, _CB, 2 * D), jnp.float32),
            pltpu.VMEM((2, _CB, 4 * D), jnp.float32),
            pltpu.VMEM((2, _CB, D), jnp.float32),
            pltpu.VMEM((2, _CB), jnp.float32),
            pltpu.VMEM((2, _CB), jnp.float32),
            pltpu.SemaphoreType.DMA,
            pltpu.SemaphoreType.DMA,
            pltpu.SemaphoreType.DMA,
            pltpu.SemaphoreType.DMA,
            pltpu.SemaphoreType.DMA,
            pltpu.SemaphoreType.DMA,
        ],
    )
    def k(tvi_hbm, tvj_hbm, tq_hbm, rel_hbm, trel_hbm, vi_hbm, vj_hbm,
          qx_hbm, out_hbm, vi_v, vj_v, qx_v, gvi, gvj, gq, relv, trelv,
          outv, semi0, semi1, semg0, semg1, semo0, semo1):
        wid = lax.axis_index("s") * _NC + lax.axis_index("c")
        base = wid * _EPT
        semi = (semi0, semi1)
        semg = (semg0, semg1)
        semo = (semo0, semo1)
        lane = lax.broadcasted_iota(jnp.int32, (16,), 0)

        def issue_idx(ci, slot):
            ci = lax.min(ci, _NCHUNK - 1)
            off = base + ci * _CB
            sem = semi[slot]
            pltpu.async_copy(vi_hbm.at[pl.ds(off, _CB)], vi_v.at[slot], sem)
            pltpu.async_copy(vj_hbm.at[pl.ds(off, _CB)], vj_v.at[slot], sem)
            pltpu.async_copy(qx_hbm.at[pl.ds(off, _CB)], qx_v.at[slot], sem)
            pltpu.async_copy(rel_hbm.at[pl.ds(off, _CB)], relv.at[slot], sem)
            pltpu.async_copy(trel_hbm.at[pl.ds(off, _CB)], trelv.at[slot],
                             sem)

        def wait_idx(slot):
            sem = semi[slot]
            z = pl.ds(0, _CB)
            pltpu.make_async_copy(vi_hbm.at[z], vi_v.at[slot], sem).wait()
            pltpu.make_async_copy(vj_hbm.at[z], vj_v.at[slot], sem).wait()
            pltpu.make_async_copy(qx_hbm.at[z], qx_v.at[slot], sem).wait()
            pltpu.make_async_copy(rel_hbm.at[z], relv.at[slot], sem).wait()
            pltpu.make_async_copy(trel_hbm.at[z], trelv.at[slot], sem).wait()

        def issue_gather(slot):
            sem = semg[slot]
            pltpu.async_copy(tvi_hbm.at[vi_v.at[slot]], gvi.at[slot], sem)
            pltpu.async_copy(tvj_hbm.at[vj_v.at[slot]], gvj.at[slot], sem)
            pltpu.async_copy(tq_hbm.at[qx_v.at[slot]], gq.at[slot], sem)

        def wait_gather(slot):
            sem = semg[slot]
            pltpu.make_async_copy(tvi_hbm.at[vi_v.at[slot]], gvi.at[slot],
                                  sem).wait()
            pltpu.make_async_copy(tvj_hbm.at[vj_v.at[slot]], gvj.at[slot],
                                  sem).wait()
            pltpu.make_async_copy(tq_hbm.at[qx_v.at[slot]], gq.at[slot],
                                  sem).wait()

        def issue_out(ci, slot):
            off = base + ci * _CB
            pltpu.async_copy(outv.at[slot], out_hbm.at[pl.ds(off, _CB)],
                             semo[slot])

        def wait_out(slot):
            pltpu.make_async_copy(outv.at[slot], out_hbm.at[pl.ds(0, _CB)],
                                  semo[slot]).wait()

        def compute(ci, slot):
            def edge(j, res):
                accs = [jnp.zeros((16,), jnp.float32) for _ in range(2)]
                for kk in range(8):
                    sl = pl.ds(kk * 16, 16)
                    u = gvi[slot, j, pl.ds(kk * 16, 16)]
                    g1 = gvi[slot, j, pl.ds(D + kk * 16, 16)]
                    mi = gvi[slot, j, pl.ds(2 * D + kk * 16, 16)]
                    mj = gvj[slot, j, pl.ds(kk * 16, 16)]
                    g2 = gvj[slot, j, pl.ds(D + kk * 16, 16)]
                    d2 = gq[slot, j, pl.ds(kk * 16, 16)]
                    c2v = gq[slot, j, pl.ds(D + kk * 16, 16)]
                    hh = gq[slot, j, pl.ds(2 * D + kk * 16, 16)]
                    r = relv[slot, j, sl]
                    t = (u * mj + mi * d2) + (c2v * mj + ((g1 + g2) + hh) * r)
                    accs[kk % 2] = accs[kk % 2] + t
                acc = (accs[0] + accs[1]) + gq[slot, j, pl.ds(3 * D, 16)]
                for st in range(4):
                    acc = acc + _shuf(acc, lane ^ (1 << st))
                return res + jnp.where(lane == j, acc, 0.0)

            res = lax.fori_loop(0, _CB, edge, trelv[slot, :])
            outv[slot, :] = res

        def half(ci, slot, first):
            other = 1 - slot
            wait_idx(other)
            issue_gather(other)
            wait_gather(slot)
            if not first:
                wait_out(slot)
            compute(ci, slot)
            issue_out(ci, slot)
            issue_idx(ci + 2, slot)

        issue_idx(0, 0)
        wait_idx(0)
        issue_gather(0)
        issue_idx(1, 1)
        half(jnp.int32(0), 0, True)
        half(jnp.int32(1), 1, True)

        def pair(t, _):
            half(2 * t, 0, False)
            half(2 * t + 1, 1, False)
            return 0

        lax.fori_loop(1, _NCHUNK // 2, pair, 0)
        half(jnp.int32(_NCHUNK - 1), 0, False)
        wait_gather(1)
        wait_idx(0)
        wait_out(0)
        wait_out(1)

    return k(tvi, tvj, tq, rel, trel, vi, vj, qx)


def kernel(edges, memorized_embedding, rel_emb, query_src_ts_emb,
           query_rel_emb, Wq, Wk):
    tvi, tvj, tq, mrr = _prep(memorized_embedding, query_src_ts_emb,
                              query_rel_emb, Wq, Wk)
    trel = _trel(rel_emb, mrr)
    vi = edges[:, 6]
    vj = edges[:, 7]
    qx = edges[:, 0]
    return _sc_edges(tvi, tvj, tq, rel_emb, trel, vi, vj, qx)


# speedup vs baseline: 1.0197x; 1.0197x over previous
import functools
import jax
import jax.numpy as jnp
from jax import lax
from jax.experimental import pallas as pl
from jax.experimental.pallas import tpu as pltpu
from jax.experimental.pallas import tpu_sc as plsc

N_NODES = 10000
N_EDGES = 320000
D = 128

NB = 2000


def _prep_body(mem_ref, qs_ref, qr_ref, wq_ref, wk_ref,
               tvi_ref, tvj_ref, tq_ref, mrr_ref):
    f32 = jnp.float32
    mem = mem_ref[...]
    qs = qs_ref[...]
    qr = qr_ref[...]

    def blk(w, i):
        return w[:, i * D:(i + 1) * D]

    Wq0, Wq1, Wq2, Wq3 = (blk(wq_ref, i) for i in range(4))
    Wk0, Wk1, Wk2, Wk3 = (blk(wk_ref, i) for i in range(4))

    def dg(a, b):
        return lax.dot_general(a, b, (((1,), (0,)), ((), ())),
                               preferred_element_type=f32)

    def dgT(a, b):
        return lax.dot_general(a, b, (((1,), (1,)), ((), ())),
                               preferred_element_type=f32)

    def ctr(a, b):
        return lax.dot_general(a, b, (((0,), (0,)), ((), ())),
                               preferred_element_type=f32)

    C = dgT(qs, Wq2) + dgT(qr, Wq3)
    Dm = dgT(qs, Wk2) + dgT(qr, Wk3)

    u = dg(mem, ctr(Wq0, Wk0))
    g1 = dg(mem, ctr(Wq0, Wk1))
    g2 = dg(mem, ctr(Wk0, Wq1))
    d2 = dg(Dm, Wq0)
    c2 = dg(C, Wk0)
    hh = dg(Dm, Wq1) + dg(C, Wk1)
    s = jnp.sum(C * Dm, axis=-1)

    col0 = (lax.broadcasted_iota(jnp.int32, (NB, D), 1) == 0).astype(f32)
    srow = s[:, None] * col0

    tvi_ref[...] = jnp.concatenate([u, g1, mem], axis=1)
    tvj_ref[...] = jnp.concatenate([mem, g2], axis=1)
    tq_ref[...] = jnp.concatenate([d2, c2, hh, srow], axis=1)
    mrr_ref[...] = ctr(Wq1, Wk1)


def _prep(mem, qs, qr, Wq, Wk):
    grid = N_NODES // NB
    return pl.pallas_call(
        _prep_body,
        grid=(grid,),
        in_specs=[
            pl.BlockSpec((NB, D), lambda i: (i, 0)),
            pl.BlockSpec((NB, D), lambda i: (i, 0)),
            pl.BlockSpec((NB, D), lambda i: (i, 0)),
            pl.BlockSpec((4 * D, 4 * D), lambda i: (0, 0)),
            pl.BlockSpec((4 * D, 4 * D), lambda i: (0, 0)),
        ],
        out_specs=[
            pl.BlockSpec((NB, 3 * D), lambda i: (i, 0)),
            pl.BlockSpec((NB, 2 * D), lambda i: (i, 0)),
            pl.BlockSpec((NB, 4 * D), lambda i: (i, 0)),
            pl.BlockSpec((D, D), lambda i: (0, 0)),
        ],
        out_shape=[
            jax.ShapeDtypeStruct((N_NODES, 3 * D), jnp.float32),
            jax.ShapeDtypeStruct((N_NODES, 2 * D), jnp.float32),
            jax.ShapeDtypeStruct((N_NODES, 4 * D), jnp.float32),
            jax.ShapeDtypeStruct((D, D), jnp.float32),
        ],
    )(mem, qs, qr, Wq, Wk)


EB = 3200


def _trel_body(rel_ref, mrr_ref, out_ref):
    rel = rel_ref[...]
    z = lax.dot_general(rel, mrr_ref[...], (((1,), (0,)), ((), ())),
                        preferred_element_type=jnp.float32)
    out_ref[...] = jnp.sum(z * rel, axis=-1)[None, None, :]


def _trel(rel, mrr):
    grid = N_EDGES // EB
    out = pl.pallas_call(
        _trel_body,
        grid=(grid,),
        in_specs=[
            pl.BlockSpec((EB, D), lambda i: (i, 0)),
            pl.BlockSpec((D, D), lambda i: (0, 0)),
        ],
        out_specs=pl.BlockSpec((1, 1, EB), lambda i: (i, 0, 0)),
        out_shape=jax.ShapeDtypeStruct((grid, 1, EB), jnp.float32),
    )(rel, mrr)
    return out.reshape(-1)


_NC, _NS = 2, 16
_NW = _NC * _NS
_EPT = N_EDGES // _NW
_CB = 16
_NCHUNK = _EPT // _CB
_DW = D // 2


def _shuf(v, idx):
    return lax.gather(
        v, idx[:, None],
        lax.GatherDimensionNumbers(offset_dims=(), collapsed_slice_dims=(0,),
                                   start_index_map=(0,)),
        slice_sizes=(1,),
        mode=lax.GatherScatterMode.PROMISE_IN_BOUNDS)


def _sc_edges(tvi, tvj, tq, rel, trel, vi, vj, qx):
    mesh = plsc.VectorSubcoreMesh(core_axis_name="c", subcore_axis_name="s")

    @functools.partial(
        pl.kernel,
        out_type=jax.ShapeDtypeStruct((N_EDGES,), jnp.float32),
        mesh=mesh,
        scratch_types=[
            pltpu.VMEM((2, _CB), jnp.int32),
            pltpu.VMEM((2, _CB), jnp.int32),
            pltpu.VMEM((2, _CB), jnp.int32),
            pltpu.VMEM((2, _CB, 3 * D), jnp.float32),
            pltpu.VMEM((2, _CB, 2 * D), jnp.float32),
            pltpu.VMEM((2, _CB, 4 * D), jnp.float32),
            pltpu.VMEM((2, _CB, D), jnp.float32),
            pltpu.VMEM((2, _CB), jnp.float32),
            pltpu.VMEM((2, _CB), jnp.float32),
            pltpu.SemaphoreType.DMA,
            pltpu.SemaphoreType.DMA,
            pltpu.SemaphoreType.DMA,
            pltpu.SemaphoreType.DMA,
            pltpu.SemaphoreType.DMA,
            pltpu.SemaphoreType.DMA,
        ],
    )
    def k(tvi_hbm, tvj_hbm, tq_hbm, rel_hbm, trel_hbm, vi_hbm, vj_hbm,
          qx_hbm, out_hbm, vi_v, vj_v, qx_v, gvi, gvj, gq, relv, trelv,
          outv, semi0, semi1, semg0, semg1, semo0, semo1):
        wid = lax.axis_index("s") * _NC + lax.axis_index("c")
        base = wid * _EPT
        semi = (semi0, semi1)
        semg = (semg0, semg1)
        semo = (semo0, semo1)
        lane = lax.broadcasted_iota(jnp.int32, (16,), 0)

        def issue_idx(ci, slot):
            ci = lax.min(ci, _NCHUNK - 1)
            off = base + ci * _CB
            sem = semi[slot]
            pltpu.async_copy(vi_hbm.at[pl.ds(off, _CB)], vi_v.at[slot], sem)
            pltpu.async_copy(vj_hbm.at[pl.ds(off, _CB)], vj_v.at[slot], sem)
            pltpu.async_copy(qx_hbm.at[pl.ds(off, _CB)], qx_v.at[slot], sem)
            pltpu.async_copy(rel_hbm.at[pl.ds(off, _CB)], relv.at[slot], sem)
            pltpu.async_copy(trel_hbm.at[pl.ds(off, _CB)], trelv.at[slot],
                             sem)

        def wait_idx(slot):
            sem = semi[slot]
            z = pl.ds(0, _CB)
            pltpu.make_async_copy(vi_hbm.at[z], vi_v.at[slot], sem).wait()
            pltpu.make_async_copy(vj_hbm.at[z], vj_v.at[slot], sem).wait()
            pltpu.make_async_copy(qx_hbm.at[z], qx_v.at[slot], sem).wait()
            pltpu.make_async_copy(rel_hbm.at[z], relv.at[slot], sem).wait()
            pltpu.make_async_copy(trel_hbm.at[z], trelv.at[slot], sem).wait()

        def issue_gather(slot):
            sem = semg[slot]
            pltpu.async_copy(tvi_hbm.at[vi_v.at[slot]], gvi.at[slot], sem)
            pltpu.async_copy(tvj_hbm.at[vj_v.at[slot]], gvj.at[slot], sem)
            pltpu.async_copy(tq_hbm.at[qx_v.at[slot]], gq.at[slot], sem)

        def wait_gather(slot):
            sem = semg[slot]
            pltpu.make_async_copy(tvi_hbm.at[vi_v.at[slot]], gvi.at[slot],
                                  sem).wait()
            pltpu.make_async_copy(tvj_hbm.at[vj_v.at[slot]], gvj.at[slot],
                                  sem).wait()
            pltpu.make_async_copy(tq_hbm.at[qx_v.at[slot]], gq.at[slot],
                                  sem).wait()

        def issue_out(ci, slot):
            off = base + ci * _CB
            pltpu.async_copy(outv.at[slot], out_hbm.at[pl.ds(off, _CB)],
                             semo[slot])

        def wait_out(slot):
            pltpu.make_async_copy(outv.at[slot], out_hbm.at[pl.ds(0, _CB)],
                                  semo[slot]).wait()

        def compute(ci, slot):
            def edge(j, res):
                accs = [jnp.zeros((16,), jnp.float32) for _ in range(2)]
                for kk in range(8):
                    sl = pl.ds(kk * 16, 16)
                    u = gvi[slot, j, pl.ds(kk * 16, 16)]
                    g1 = gvi[slot, j, pl.ds(D + kk * 16, 16)]
                    mi = gvi[slot, j, pl.ds(2 * D + kk * 16, 16)]
                    mj = gvj[slot, j, pl.ds(kk * 16, 16)]
                    g2 = gvj[slot, j, pl.ds(D + kk * 16, 16)]
                    d2 = gq[slot, j, pl.ds(kk * 16, 16)]
                    c2v = gq[slot, j, pl.ds(D + kk * 16, 16)]
                    hh = gq[slot, j, pl.ds(2 * D + kk * 16, 16)]
                    r = relv[slot, j, sl]
                    t = (u * mj + mi * d2) + (c2v * mj + ((g1 + g2) + hh) * r)
                    accs[kk % 2] = accs[kk % 2] + t
                acc = (accs[0] + accs[1]) + gq[slot, j, pl.ds(3 * D, 16)]
                for st in range(4):
                    acc = acc + _shuf(acc, lane ^ (1 << st))
                return res + jnp.where(lane == j, acc, 0.0)

            res = lax.fori_loop(0, _CB, edge, trelv[slot, :])
            outv[slot, :] = res

        def half(ci, slot, first):
            other = 1 - slot
            wait_idx(other)
            issue_gather(other)
            wait_gather(slot)
            if not first:
                wait_out(slot)
            compute(ci, slot)
            issue_out(ci, slot)
            issue_idx(ci + 2, slot)

        issue_idx(0, 0)
        wait_idx(0)
        issue_gather(0)
        issue_idx(1, 1)
        half(jnp.int32(0), 0, True)
        half(jnp.int32(1), 1, True)

        def pair(t, _):
            half(2 * t, 0, False)
            half(2 * t + 1, 1, False)
            return 0

        lax.fori_loop(1, _NCHUNK // 2, pair, 0)
        half(jnp.int32(_NCHUNK - 1), 0, False)
        wait_gather(1)
        wait_idx(0)
        wait_out(0)
        wait_out(1)

    return k(tvi, tvj, tq, rel, trel, vi, vj, qx)


def kernel(edges, memorized_embedding, rel_emb, query_src_ts_emb,
           query_rel_emb, Wq, Wk):
    tvi, tvj, tq, mrr = _prep(memorized_embedding, query_src_ts_emb,
                              query_rel_emb, Wq, Wk)
    trel = _trel(rel_emb, mrr)
    vi = edges[:, 6]
    vj = edges[:, 7]
    qx = edges[:, 0]
    return _sc_edges(tvi, tvj, tq, rel_emb, trel, vi, vj, qx)


# speedup vs baseline: 1.0368x; 1.0167x over previous
import functools
import jax
import jax.numpy as jnp
from jax import lax
from jax.experimental import pallas as pl
from jax.experimental.pallas import tpu as pltpu
from jax.experimental.pallas import tpu_sc as plsc

N_NODES = 10000
N_EDGES = 320000
D = 128

NB = 2000


def _prep_body(mem_ref, qs_ref, qr_ref, wq_ref, wk_ref,
               tvi_ref, tvj_ref, tq_ref, mrr_ref):
    f32 = jnp.float32
    mem = mem_ref[...]
    qs = qs_ref[...]
    qr = qr_ref[...]

    def blk(w, i):
        return w[:, i * D:(i + 1) * D]

    Wq0, Wq1, Wq2, Wq3 = (blk(wq_ref, i) for i in range(4))
    Wk0, Wk1, Wk2, Wk3 = (blk(wk_ref, i) for i in range(4))

    def dg(a, b):
        return lax.dot_general(a, b, (((1,), (0,)), ((), ())),
                               preferred_element_type=f32)

    def dgT(a, b):
        return lax.dot_general(a, b, (((1,), (1,)), ((), ())),
                               preferred_element_type=f32)

    def ctr(a, b):
        return lax.dot_general(a, b, (((0,), (0,)), ((), ())),
                               preferred_element_type=f32)

    C = dgT(qs, Wq2) + dgT(qr, Wq3)
    Dm = dgT(qs, Wk2) + dgT(qr, Wk3)

    u = dg(mem, ctr(Wq0, Wk0))
    g1 = dg(mem, ctr(Wq0, Wk1))
    g2 = dg(mem, ctr(Wk0, Wq1))
    d2 = dg(Dm, Wq0)
    c2 = dg(C, Wk0)
    hh = dg(Dm, Wq1) + dg(C, Wk1)
    s = jnp.sum(C * Dm, axis=-1)

    col0 = (lax.broadcasted_iota(jnp.int32, (NB, D), 1) == 0).astype(f32)
    srow = s[:, None] * col0

    tvi_ref[...] = jnp.concatenate([u, g1, mem], axis=1)
    tvj_ref[...] = jnp.concatenate([mem, g2], axis=1)
    tq_ref[...] = jnp.concatenate([d2, c2, hh, srow], axis=1)
    mrr_ref[...] = ctr(Wq1, Wk1)


def _prep(mem, qs, qr, Wq, Wk):
    grid = N_NODES // NB
    return pl.pallas_call(
        _prep_body,
        grid=(grid,),
        in_specs=[
            pl.BlockSpec((NB, D), lambda i: (i, 0)),
            pl.BlockSpec((NB, D), lambda i: (i, 0)),
            pl.BlockSpec((NB, D), lambda i: (i, 0)),
            pl.BlockSpec((4 * D, 4 * D), lambda i: (0, 0)),
            pl.BlockSpec((4 * D, 4 * D), lambda i: (0, 0)),
        ],
        out_specs=[
            pl.BlockSpec((NB, 3 * D), lambda i: (i, 0)),
            pl.BlockSpec((NB, 2 * D), lambda i: (i, 0)),
            pl.BlockSpec((NB, 4 * D), lambda i: (i, 0)),
            pl.BlockSpec((D, D), lambda i: (0, 0)),
        ],
        out_shape=[
            jax.ShapeDtypeStruct((N_NODES, 3 * D), jnp.float32),
            jax.ShapeDtypeStruct((N_NODES, 2 * D), jnp.float32),
            jax.ShapeDtypeStruct((N_NODES, 4 * D), jnp.float32),
            jax.ShapeDtypeStruct((D, D), jnp.float32),
        ],
    )(mem, qs, qr, Wq, Wk)


EB = 3200


def _trel_body(rel_ref, mrr_ref, out_ref):
    rel = rel_ref[...]
    z = lax.dot_general(rel, mrr_ref[...], (((1,), (0,)), ((), ())),
                        preferred_element_type=jnp.float32)
    out_ref[...] = jnp.sum(z * rel, axis=-1)[None, None, :]


def _trel(rel, mrr):
    grid = N_EDGES // EB
    out = pl.pallas_call(
        _trel_body,
        grid=(grid,),
        in_specs=[
            pl.BlockSpec((EB, D), lambda i: (i, 0)),
            pl.BlockSpec((D, D), lambda i: (0, 0)),
        ],
        out_specs=pl.BlockSpec((1, 1, EB), lambda i: (i, 0, 0)),
        out_shape=jax.ShapeDtypeStruct((grid, 1, EB), jnp.float32),
    )(rel, mrr)
    return out.reshape(-1)


_NC, _NS = 2, 16
_NW = _NC * _NS
_EPT = N_EDGES // _NW
_CB = 16
_NCHUNK = _EPT // _CB
_DW = D // 2


def _shuf(v, idx):
    return lax.gather(
        v, idx[:, None],
        lax.GatherDimensionNumbers(offset_dims=(), collapsed_slice_dims=(0,),
                                   start_index_map=(0,)),
        slice_sizes=(1,),
        mode=lax.GatherScatterMode.PROMISE_IN_BOUNDS)


def _sc_edges(tvi, tvj, tq, rel, trel, vi, vj, qx):
    mesh = plsc.VectorSubcoreMesh(core_axis_name="c", subcore_axis_name="s")

    @functools.partial(
        pl.kernel,
        out_type=jax.ShapeDtypeStruct((N_EDGES,), jnp.float32),
        mesh=mesh,
        scratch_types=[
            pltpu.VMEM((2, _CB), jnp.int32),
            pltpu.VMEM((2, _CB), jnp.int32),
            pltpu.VMEM((2, _CB), jnp.int32),
            pltpu.VMEM((2, _CB, 3 * D), jnp.float32),
            pltpu.VMEM((2, _CB, 2 * D), jnp.float32),
            pltpu.VMEM((2, _CB, 4 * D), jnp.float32),
            pltpu.VMEM((2, _CB, D), jnp.float32),
            pltpu.VMEM((2, _CB), jnp.float32),
            pltpu.VMEM((2, _CB), jnp.float32),
            pltpu.SemaphoreType.DMA,
            pltpu.SemaphoreType.DMA,
            pltpu.SemaphoreType.DMA,
            pltpu.SemaphoreType.DMA,
            pltpu.SemaphoreType.DMA,
            pltpu.SemaphoreType.DMA,
        ],
    )
    def k(tvi_hbm, tvj_hbm, tq_hbm, rel_hbm, trel_hbm, vi_hbm, vj_hbm,
          qx_hbm, out_hbm, vi_v, vj_v, qx_v, gvi, gvj, gq, relv, trelv,
          outv, semi0, semi1, semg0, semg1, semo0, semo1):
        wid = lax.axis_index("s") * _NC + lax.axis_index("c")
        base = wid * _EPT
        semi = (semi0, semi1)
        semg = (semg0, semg1)
        semo = (semo0, semo1)
        lane = lax.broadcasted_iota(jnp.int32, (16,), 0)

        def issue_idx(ci, slot):
            ci = lax.min(ci, _NCHUNK - 1)
            off = base + ci * _CB
            sem = semi[slot]
            pltpu.async_copy(vi_hbm.at[pl.ds(off, _CB)], vi_v.at[slot], sem)
            pltpu.async_copy(vj_hbm.at[pl.ds(off, _CB)], vj_v.at[slot], sem)
            pltpu.async_copy(qx_hbm.at[pl.ds(off, _CB)], qx_v.at[slot], sem)
            pltpu.async_copy(rel_hbm.at[pl.ds(off, _CB)], relv.at[slot], sem)
            pltpu.async_copy(trel_hbm.at[pl.ds(off, _CB)], trelv.at[slot],
                             sem)

        def wait_idx(slot):
            sem = semi[slot]
            z = pl.ds(0, _CB)
            pltpu.make_async_copy(vi_hbm.at[z], vi_v.at[slot], sem).wait()
            pltpu.make_async_copy(vj_hbm.at[z], vj_v.at[slot], sem).wait()
            pltpu.make_async_copy(qx_hbm.at[z], qx_v.at[slot], sem).wait()
            pltpu.make_async_copy(rel_hbm.at[z], relv.at[slot], sem).wait()
            pltpu.make_async_copy(trel_hbm.at[z], trelv.at[slot], sem).wait()

        def issue_gather(slot):
            sem = semg[slot]
            pltpu.async_copy(tvi_hbm.at[vi_v.at[slot]], gvi.at[slot], sem)
            pltpu.async_copy(tvj_hbm.at[vj_v.at[slot]], gvj.at[slot], sem)
            pltpu.async_copy(tq_hbm.at[qx_v.at[slot]], gq.at[slot], sem)

        def wait_gather(slot):
            sem = semg[slot]
            pltpu.make_async_copy(tvi_hbm.at[vi_v.at[slot]], gvi.at[slot],
                                  sem).wait()
            pltpu.make_async_copy(tvj_hbm.at[vj_v.at[slot]], gvj.at[slot],
                                  sem).wait()
            pltpu.make_async_copy(tq_hbm.at[qx_v.at[slot]], gq.at[slot],
                                  sem).wait()

        def issue_out(ci, slot):
            off = base + ci * _CB
            pltpu.async_copy(outv.at[slot], out_hbm.at[pl.ds(off, _CB)],
                             semo[slot])

        def wait_out(slot):
            pltpu.make_async_copy(outv.at[slot], out_hbm.at[pl.ds(0, _CB)],
                                  semo[slot]).wait()

        def compute(ci, slot):
            def edge(j, res):
                acc = jnp.zeros((16,), jnp.float32)
                for kk in range(8):
                    sl = pl.ds(kk * 16, 16)
                    u = gvi[slot, j, pl.ds(kk * 16, 16)]
                    g1 = gvi[slot, j, pl.ds(D + kk * 16, 16)]
                    mi = gvi[slot, j, pl.ds(2 * D + kk * 16, 16)]
                    mj = gvj[slot, j, pl.ds(kk * 16, 16)]
                    g2 = gvj[slot, j, pl.ds(D + kk * 16, 16)]
                    d2 = gq[slot, j, pl.ds(kk * 16, 16)]
                    c2v = gq[slot, j, pl.ds(D + kk * 16, 16)]
                    hh = gq[slot, j, pl.ds(2 * D + kk * 16, 16)]
                    r = relv[slot, j, sl]
                    acc = (acc + u * mj + mi * d2 + c2v * mj
                           + (g1 + g2 + hh) * r)
                acc = acc + gq[slot, j, pl.ds(3 * D, 16)]
                for st in range(4):
                    acc = acc + _shuf(acc, lane ^ (1 << st))
                return res + jnp.where(lane == j, acc, 0.0)

            res = lax.fori_loop(0, _CB, edge, trelv[slot, :])
            outv[slot, :] = res

        def half(ci, slot, first):
            other = 1 - slot
            wait_idx(other)
            issue_gather(other)
            wait_gather(slot)
            if not first:
                wait_out(slot)
            compute(ci, slot)
            issue_out(ci, slot)
            issue_idx(ci + 2, slot)

        issue_idx(0, 0)
        wait_idx(0)
        issue_gather(0)
        issue_idx(1, 1)
        half(jnp.int32(0), 0, True)
        half(jnp.int32(1), 1, True)

        def pair(t, _):
            half(2 * t, 0, False)
            half(2 * t + 1, 1, False)
            return 0

        lax.fori_loop(1, _NCHUNK // 2, pair, 0)
        half(jnp.int32(_NCHUNK - 1), 0, False)
        wait_gather(1)
        wait_idx(0)
        wait_out(0)
        wait_out(1)

    return k(tvi, tvj, tq, rel, trel, vi, vj, qx)


def kernel(edges, memorized_embedding, rel_emb, query_src_ts_emb,
           query_rel_emb, Wq, Wk):
    tvi, tvj, tq, mrr = _prep(memorized_embedding, query_src_ts_emb,
                              query_rel_emb, Wq, Wk)
    trel = _trel(rel_emb, mrr)
    vi = edges[:, 6]
    vj = edges[:, 7]
    qx = edges[:, 0]
    return _sc_edges(tvi, tvj, tq, rel_emb, trel, vi, vj, qx)
